# Initial kernel scaffold; baseline (speedup 1.0000x reference)
import jax
import jax.numpy as jnp
from jax.experimental import pallas as pl


def kernel(embeddings, targets, contexts, negatives):
    raise NotImplementedError("write your pallas kernel here")


# speedup vs baseline: 7.2950x; 7.2950x over previous
import functools

import jax
import jax.numpy as jnp
from jax import lax
from jax.experimental import pallas as pl
from jax.experimental.pallas import tpu as pltpu
from jax.experimental.pallas import tpu_sc as plsc

NUM_NODES = 100000
DIM = 128
BATCH = 16384
NUM_NEG = 200
NPAD = 208
SCALE = 5.0

NC, NS, L = 2, 16, 16
NW = NC * NS
BPW = BATCH // NW
KD = DIM // L


def _sc_scores():
    mesh = plsc.VectorSubcoreMesh(core_axis_name="c", subcore_axis_name="s")

    @functools.partial(
        pl.kernel,
        mesh=mesh,
        compiler_params=pltpu.CompilerParams(needs_layout_passes=False),
        out_type=[
            jax.ShapeDtypeStruct((BATCH * L,), jnp.float32),
            jax.ShapeDtypeStruct((BATCH * NPAD,), jnp.float32),
        ],
        scratch_types=[
            pltpu.VMEM((BPW,), jnp.int32),
            pltpu.VMEM((BPW,), jnp.int32),
            pltpu.VMEM((BPW, DIM), jnp.float32),
            pltpu.VMEM((2, NPAD, DIM), jnp.float32),
            pltpu.VMEM((NPAD,), jnp.int32),
            pltpu.VMEM((NPAD,), jnp.int32),
            pltpu.VMEM((NPAD,), jnp.float32),
            pltpu.VMEM((NPAD,), jnp.float32),
            pltpu.VMEM((BPW // 4 * L,), jnp.float32),
            pltpu.SemaphoreType.DMA,
            pltpu.SemaphoreType.DMA,
        ],
    )
    def sc_fn(emb, tgt, ctx, neg, pos_out, neg_out,
              tidx, cidx, trows, nrows, nidx0, nidx1, sbuf0, sbuf1, pstage,
              sem0, sem1):
        wid = lax.axis_index("s") * NC + lax.axis_index("c")
        base = wid * BPW
        iota = lax.iota(jnp.int32, L)

        pltpu.sync_copy(tgt.at[pl.ds(base, BPW)], tidx)
        pltpu.sync_copy(ctx.at[pl.ds(base, BPW)], cidx)

        cb = BPW // 4
        for c in range(4):
            h0 = pltpu.async_copy(emb.at[tidx.at[pl.ds(c * cb, cb)]],
                                  nrows.at[0, pl.ds(0, cb)], sem0)
            h1 = pltpu.async_copy(emb.at[cidx.at[pl.ds(c * cb, cb)]],
                                  nrows.at[1, pl.ds(0, cb)], sem1)
            h0.wait()
            h1.wait()

            def pos_body(ii, carry):
                acc = jnp.zeros((L,), jnp.float32)
                for k in range(KD):
                    tv = nrows[0, ii, pl.ds(k * L, L)]
                    cv = nrows[1, ii, pl.ds(k * L, L)]
                    acc = acc + tv * cv
                pstage[pl.ds(ii * L, L)] = acc
                return carry

            lax.fori_loop(0, cb, pos_body, 0)
            pltpu.sync_copy(pstage,
                            pos_out.at[pl.ds((base + c * cb) * L, cb * L)])

        for c in range(4):
            pltpu.async_copy(emb.at[tidx.at[pl.ds(c * cb, cb)]],
                             trows.at[pl.ds(c * cb, cb)], sem0).wait()

        def item_body(i, carry):
            b = base + i
            pltpu.sync_copy(neg.at[pl.ds(b * NUM_NEG, NUM_NEG)],
                            nidx0.at[pl.ds(0, NUM_NEG)])
            g0 = pltpu.async_copy(emb.at[nidx0.at[pl.ds(0, 128)]],
                                  nrows.at[0, pl.ds(0, 128)], sem0)
            g1 = pltpu.async_copy(emb.at[nidx0.at[pl.ds(128, NUM_NEG - 128)]],
                                  nrows.at[0, pl.ds(128, NUM_NEG - 128)], sem1)
            g0.wait()
            g1.wait()

            tvecs = [trows[i, pl.ds(k * L, L)] for k in range(KD)]

            def group_body(g, carry2):
                svec = jnp.zeros((L,), jnp.float32)
                for j in range(L):
                    r = g * L + j
                    acc = nrows[0, r, pl.ds(0, L)] * tvecs[0]
                    for k in range(1, KD):
                        acc = acc + nrows[0, r, pl.ds(k * L, L)] * tvecs[k]
                    svec = jnp.where(iota == j, jnp.sum(acc), svec)
                sbuf0[pl.ds(g * L, L)] = svec
                return carry2

            lax.fori_loop(0, NPAD // L, group_body, 0)
            pltpu.sync_copy(sbuf0, neg_out.at[pl.ds(b * NPAD, NPAD)])
            return carry

        lax.fori_loop(0, BPW, item_body, 0)

    return sc_fn


_SC_SCORES = _sc_scores()


def _loss_kernel(pos_ref, neg_ref, out_ref):
    i = pl.program_id(0)
    pos_s = jnp.sum(pos_ref[...], axis=1, keepdims=True) * SCALE
    neg_s = neg_ref[...] * SCALE

    def logsig(x):
        return jnp.minimum(x, 0.0) - jnp.log1p(jnp.exp(-jnp.abs(x)))

    col = lax.broadcasted_iota(jnp.int32, neg_s.shape, 1)
    neg_l = jnp.where(col < NUM_NEG, logsig(-neg_s), 0.0)
    partial = (-jnp.sum(logsig(pos_s)) / BATCH
               - jnp.sum(neg_l) / (BATCH * NUM_NEG))

    @pl.when(i == 0)
    def _():
        out_ref[0, 0] = 0.0

    out_ref[0, 0] += partial


def kernel(embeddings, targets, contexts, negatives):
    pos_partial, neg_scores = _SC_SCORES(embeddings, targets, contexts,
                                         negatives.reshape(-1))
    pos_partial = pos_partial.reshape(BATCH, L)
    neg_scores = neg_scores.reshape(BATCH, NPAD)
    rows = BATCH // 16
    loss = pl.pallas_call(
        _loss_kernel,
        grid=(16,),
        in_specs=[
            pl.BlockSpec((rows, L), lambda i: (i, 0)),
            pl.BlockSpec((rows, NPAD), lambda i: (i, 0)),
        ],
        out_specs=pl.BlockSpec(memory_space=pltpu.SMEM),
        out_shape=jax.ShapeDtypeStruct((1, 1), jnp.float32),
    )(pos_partial, neg_scores)
    return loss[0, 0]
